# Initial kernel scaffold; baseline (speedup 1.0000x reference)
import jax
import jax.numpy as jnp
from jax.experimental import pallas as pl


def kernel(spans, we):
    raise NotImplementedError("write your pallas kernel here")


# speedup vs baseline: 12.6793x; 12.6793x over previous
import functools

import jax
import jax.numpy as jnp
from jax import lax
from jax.experimental import pallas as pl
from jax.experimental.pallas import tpu as pltpu
from jax.experimental.pallas import tpu_sc as plsc

B = 16384
L = 50
D = 64
NC = 2
NS = 16
NW = NC * NS
SPT = B // NW
S = 8
NCHUNK = SPT // S
NLANE = 16
NREG = D // NLANE


def _body(spans_hbm, we_hbm, out_hbm, idx_v, rows_v, out_v, sem):
    wid = lax.axis_index("s") * NC + lax.axis_index("c")
    base = wid * SPT

    @pl.loop(0, NCHUNK)
    def _chunk(c):
        cb = base + c * S
        pltpu.sync_copy(spans_hbm.at[pl.ds(cb * L, S * L)], idx_v)
        pltpu.async_copy(we_hbm.at[idx_v], rows_v, sem).wait()
        for s in range(S):
            accs = tuple(jnp.zeros((NLANE,), jnp.float32) for _ in range(NREG))

            def _acc(j, accs, s=s):
                return tuple(
                    accs[d] + rows_v[s * L + j, pl.ds(d * NLANE, NLANE)]
                    for d in range(NREG)
                )

            accs = lax.fori_loop(0, L, _acc, accs)
            for d in range(NREG):
                out_v[s, pl.ds(d * NLANE, NLANE)] = accs[d]
        pltpu.sync_copy(out_v, out_hbm.at[pl.ds(cb, S), :])


@jax.jit
def kernel(spans, we):
    spans_flat = spans.reshape(-1).astype(jnp.int32)
    mesh = plsc.VectorSubcoreMesh(
        core_axis_name="c", subcore_axis_name="s", num_cores=NC, num_subcores=NS
    )
    f = pl.kernel(
        _body,
        out_type=jax.ShapeDtypeStruct((B, D), jnp.float32),
        mesh=mesh,
        scratch_types=[
            pltpu.VMEM((S * L,), jnp.int32),
            pltpu.VMEM((S * L, D), jnp.float32),
            pltpu.VMEM((S, D), jnp.float32),
            pltpu.SemaphoreType.DMA,
        ],
        compiler_params=pltpu.CompilerParams(use_tc_tiling_on_sc=False),
    )
    return f(spans_flat, we)


# speedup vs baseline: 20.5845x; 1.6235x over previous
import functools

import jax
import jax.numpy as jnp
from jax import lax
from jax.experimental import pallas as pl
from jax.experimental.pallas import tpu as pltpu
from jax.experimental.pallas import tpu_sc as plsc

B = 16384
L = 50
D = 64
NC = 2
NS = 16
NW = NC * NS
SPT = B // NW
S = 8
NCHUNK = SPT // S
NLANE = 16
NREG = D // NLANE


def _body(spans_hbm, we_hbm, out_hbm, idx_all, rows0, rows1, out_v, sem0, sem1):
    wid = lax.axis_index("s") * NC + lax.axis_index("c")
    base = wid * SPT
    rows = (rows0, rows1)
    sems = (sem0, sem1)

    pltpu.sync_copy(spans_hbm.at[pl.ds(base * L, SPT * L)], idx_all)

    def gather_start(c, b):
        pltpu.async_copy(
            we_hbm.at[idx_all.at[pl.ds(c * (S * L), S * L)]], rows[b], sems[b]
        )

    def gather_wait(c, b):
        pltpu.make_async_copy(
            we_hbm.at[idx_all.at[pl.ds(c * (S * L), S * L)]], rows[b], sems[b]
        ).wait()

    gather_start(0, 0)

    @pl.loop(0, NCHUNK, step=2)
    def _pair(c):
        for b in range(2):
            cc = c + b

            @pl.when(cc + 1 < NCHUNK)
            def _():
                gather_start(cc + 1, 1 - b)

            gather_wait(cc, b)
            rv = rows[b]
            for s in range(S):
                accs = tuple(jnp.zeros((NLANE,), jnp.float32) for _ in range(NREG))

                def _acc(j, accs, s=s, rv=rv):
                    return tuple(
                        accs[d] + rv[s * L + j, pl.ds(d * NLANE, NLANE)]
                        for d in range(NREG)
                    )

                accs = lax.fori_loop(0, L, _acc, accs, unroll=10)
                for d in range(NREG):
                    out_v[s, pl.ds(d * NLANE, NLANE)] = accs[d]
            pltpu.sync_copy(out_v, out_hbm.at[pl.ds(base + cc * S, S), :])


@jax.jit
def kernel(spans, we):
    spans_flat = spans.reshape(-1).astype(jnp.int32)
    mesh = plsc.VectorSubcoreMesh(
        core_axis_name="c", subcore_axis_name="s", num_cores=NC, num_subcores=NS
    )
    f = pl.kernel(
        _body,
        out_type=jax.ShapeDtypeStruct((B, D), jnp.float32),
        mesh=mesh,
        scratch_types=[
            pltpu.VMEM((SPT * L,), jnp.int32),
            pltpu.VMEM((S * L, D), jnp.float32),
            pltpu.VMEM((S * L, D), jnp.float32),
            pltpu.VMEM((S, D), jnp.float32),
            pltpu.SemaphoreType.DMA,
            pltpu.SemaphoreType.DMA,
        ],
        compiler_params=pltpu.CompilerParams(use_tc_tiling_on_sc=False),
    )
    return f(spans_flat, we)
